# baseline (device time: 119916 ns/iter reference)
import jax
import jax.numpy as jnp
from jax import lax
from jax.experimental import pallas as pl
from jax.experimental.pallas import tpu as pltpu

N_DEV = 4
EPS = 1e-5

_sem_signal = getattr(pltpu, "semaphore_signal", None) or pl.semaphore_signal
_sem_wait = getattr(pltpu, "semaphore_wait", None) or pl.semaphore_wait
_DevId = getattr(pltpu, "DeviceIdType", None) or pl.DeviceIdType

B, H_LOC, W, C = 2, 256, 256, 128
C_OUT = 2 * C
N_GLOBAL = float(N_DEV * H_LOC * W)

HB_STATS = 64
HB_APPLY = 32
NJ_STATS = H_LOC // HB_STATS
NJ_APPLY = H_LOC // HB_APPLY


def _stats_body(x_ref, out_ref):
    b = pl.program_id(0)
    j = pl.program_id(1)
    step = b * NJ_STATS + j

    @pl.when(step == 0)
    def _():
        out_ref[...] = jnp.zeros_like(out_ref)

    xb = x_ref[0]
    s = jnp.sum(jnp.sum(xb, axis=0), axis=0)
    ss = jnp.sum(jnp.sum(xb * xb, axis=0), axis=0)

    @pl.when(b == 0)
    def _():
        out_ref[0, :] += s
        out_ref[2, :] += ss

    @pl.when(b == 1)
    def _():
        out_ref[1, :] += s
        out_ref[3, :] += ss


def _apply_body(x_ref, stats_ref, wp_ref, out_ref,
                mr_ref, comm_ref, send_sems, recv_sems):
    b = pl.program_id(0)
    j = pl.program_id(1)
    step = b * NJ_APPLY + j

    @pl.when(step == 0)
    def _():
        my = lax.axis_index("i")
        left = lax.rem(my - 1 + N_DEV, N_DEV)
        right = lax.rem(my + 1, N_DEV)

        barrier = pltpu.get_barrier_semaphore()
        for nbr in (left, right):
            _sem_signal(barrier, inc=1, device_id=(nbr,),
                        device_id_type=_DevId.MESH)
        _sem_wait(barrier, 2)

        comm_ref[0] = stats_ref[...]
        tot = stats_ref[...]
        for h in range(N_DEV - 1):
            rdma = pltpu.make_async_remote_copy(
                src_ref=comm_ref.at[h],
                dst_ref=comm_ref.at[h + 1],
                send_sem=send_sems.at[h],
                recv_sem=recv_sems.at[h + 1],
                device_id=(right,),
                device_id_type=_DevId.MESH,
            )
            rdma.start()
            rdma.wait()
            tot = tot + comm_ref[h + 1]

        mean = tot[0:2, :] * (1.0 / N_GLOBAL)
        ex2 = tot[2:4, :] * (1.0 / N_GLOBAL)
        var = ex2 - mean * mean
        rstd = lax.rsqrt(var + EPS)
        mr_ref[0:2, :] = mean
        mr_ref[2:4, :] = rstd
        mr_ref[4:8, :] = jnp.zeros((4, C), jnp.float32)

    mean = jnp.where(b == 0, mr_ref[0, :], mr_ref[1, :])
    rstd = jnp.where(b == 0, mr_ref[2, :], mr_ref[3, :])

    xb = x_ref[0]
    h = (xb - mean[None, None, :]) * rstd[None, None, :]
    a = h / (1.0 + jnp.exp(-h))
    flat = a.reshape(HB_APPLY * W, C).astype(jnp.bfloat16)
    wp = wp_ref[...].astype(jnp.bfloat16)
    out = jnp.dot(flat, wp, preferred_element_type=jnp.float32)
    out_ref[0] = out.reshape(HB_APPLY, W, C_OUT).astype(jnp.bfloat16)


def kernel(x, Wp):
    partial = pl.pallas_call(
        _stats_body,
        grid=(B, NJ_STATS),
        in_specs=[
            pl.BlockSpec((1, HB_STATS, W, C), lambda b, j: (b, j, 0, 0)),
        ],
        out_specs=pl.BlockSpec((8, C), lambda b, j: (0, 0)),
        out_shape=jax.ShapeDtypeStruct((8, C), jnp.float32),
        compiler_params=pltpu.CompilerParams(
            dimension_semantics=("arbitrary", "arbitrary"),
        ),
    )(x)

    out = pl.pallas_call(
        _apply_body,
        grid=(B, NJ_APPLY),
        in_specs=[
            pl.BlockSpec((1, HB_APPLY, W, C), lambda b, j: (b, j, 0, 0)),
            pl.BlockSpec((8, C), lambda b, j: (0, 0)),
            pl.BlockSpec((C, C_OUT), lambda b, j: (0, 0)),
        ],
        out_specs=pl.BlockSpec((1, HB_APPLY, W, C_OUT), lambda b, j: (b, j, 0, 0)),
        out_shape=jax.ShapeDtypeStruct((B, H_LOC, W, C_OUT), jnp.bfloat16),
        scratch_shapes=[
            pltpu.VMEM((8, C), jnp.float32),
            pltpu.VMEM((N_DEV, 8, C), jnp.float32),
            pltpu.SemaphoreType.DMA((N_DEV,)),
            pltpu.SemaphoreType.DMA((N_DEV,)),
        ],
        compiler_params=pltpu.CompilerParams(
            dimension_semantics=("arbitrary", "arbitrary"),
            collective_id=0,
        ),
    )(x, partial, Wp)
    return out


# device time: 77580 ns/iter; 1.5457x vs baseline; 1.5457x over previous
import jax
import jax.numpy as jnp
from jax import lax
from jax.experimental import pallas as pl
from jax.experimental.pallas import tpu as pltpu

N_DEV = 4
EPS = 1e-5

_sem_signal = getattr(pltpu, "semaphore_signal", None) or pl.semaphore_signal
_sem_wait = getattr(pltpu, "semaphore_wait", None) or pl.semaphore_wait
_DevId = getattr(pltpu, "DeviceIdType", None) or pl.DeviceIdType

B, H_LOC, W, C = 2, 256, 256, 128
C_OUT = 2 * C
N_GLOBAL = float(N_DEV * H_LOC * W)

HB_STATS = 64
HB_APPLY = 32
NJ_STATS = H_LOC // HB_STATS
NJ_APPLY = H_LOC // HB_APPLY


def _stats_body(x_ref, out_ref, tot_ref, comm_ref, send_sems, recv_sems):
    b = pl.program_id(0)
    j = pl.program_id(1)
    step = b * NJ_STATS + j
    last = B * NJ_STATS - 1

    my = lax.axis_index("i")
    left = lax.rem(my - 1 + N_DEV, N_DEV)
    right = lax.rem(my + 1, N_DEV)

    @pl.when(step == 0)
    def _():
        tot_ref[...] = jnp.zeros_like(tot_ref)
        barrier = pltpu.get_barrier_semaphore()
        for nbr in (left, right):
            _sem_signal(barrier, inc=1, device_id=(nbr,),
                        device_id_type=_DevId.MESH)
        _sem_wait(barrier, 2)

    xb = x_ref[0]
    s = jnp.sum(jnp.sum(xb, axis=0), axis=0)
    ss = jnp.sum(jnp.sum(xb * xb, axis=0), axis=0)

    @pl.when(b == 0)
    def _():
        tot_ref[0, :] += s
        tot_ref[2, :] += ss

    @pl.when(b == 1)
    def _():
        tot_ref[1, :] += s
        tot_ref[3, :] += ss

    @pl.when(step == last)
    def _():
        comm_ref[0] = tot_ref[...]
        for h in range(N_DEV - 1):
            rdma = pltpu.make_async_remote_copy(
                src_ref=comm_ref.at[h],
                dst_ref=comm_ref.at[h + 1],
                send_sem=send_sems.at[h],
                recv_sem=recv_sems.at[h + 1],
                device_id=(right,),
                device_id_type=_DevId.MESH,
            )
            rdma.start()
            rdma.wait()
            tot_ref[...] += comm_ref[h + 1]

        tot = tot_ref[...]
        mean = tot[0:2, :] * (1.0 / N_GLOBAL)
        ex2 = tot[2:4, :] * (1.0 / N_GLOBAL)
        var = ex2 - mean * mean
        rstd = lax.rsqrt(var + EPS)
        out_ref[0:2, :] = mean
        out_ref[2:4, :] = rstd
        out_ref[4:8, :] = jnp.zeros((4, C), jnp.float32)


def _apply_body(x_ref, stats_ref, wp_ref, out_ref):
    b = pl.program_id(0)
    mean = jnp.where(b == 0, stats_ref[0, :], stats_ref[1, :])
    rstd = jnp.where(b == 0, stats_ref[2, :], stats_ref[3, :])

    xb = x_ref[0]
    h = (xb - mean[None, None, :]) * rstd[None, None, :]
    a = h / (1.0 + jnp.exp(-h))
    flat = a.reshape(HB_APPLY * W, C).astype(jnp.bfloat16)
    wp = wp_ref[...].astype(jnp.bfloat16)
    out = jnp.dot(flat, wp, preferred_element_type=jnp.float32)
    out_ref[0] = out.reshape(HB_APPLY, W, C_OUT).astype(jnp.bfloat16)


def kernel(x, Wp):
    stats = pl.pallas_call(
        _stats_body,
        grid=(B, NJ_STATS),
        in_specs=[
            pl.BlockSpec((1, HB_STATS, W, C), lambda b, j: (b, j, 0, 0)),
        ],
        out_specs=pl.BlockSpec((8, C), lambda b, j: (0, 0)),
        out_shape=jax.ShapeDtypeStruct((8, C), jnp.float32),
        scratch_shapes=[
            pltpu.VMEM((8, C), jnp.float32),
            pltpu.VMEM((N_DEV, 8, C), jnp.float32),
            pltpu.SemaphoreType.DMA((N_DEV,)),
            pltpu.SemaphoreType.DMA((N_DEV,)),
        ],
        compiler_params=pltpu.CompilerParams(
            dimension_semantics=("arbitrary", "arbitrary"),
            collective_id=0,
        ),
    )(x)

    out = pl.pallas_call(
        _apply_body,
        grid=(B, NJ_APPLY),
        in_specs=[
            pl.BlockSpec((1, HB_APPLY, W, C), lambda b, j: (b, j, 0, 0)),
            pl.BlockSpec((8, C), lambda b, j: (0, 0)),
            pl.BlockSpec((C, C_OUT), lambda b, j: (0, 0)),
        ],
        out_specs=pl.BlockSpec((1, HB_APPLY, W, C_OUT), lambda b, j: (b, j, 0, 0)),
        out_shape=jax.ShapeDtypeStruct((B, H_LOC, W, C_OUT), jnp.bfloat16),
        compiler_params=pltpu.CompilerParams(
            dimension_semantics=("arbitrary", "arbitrary"),
        ),
    )(x, stats, Wp)
    return out


# device time: 72590 ns/iter; 1.6520x vs baseline; 1.0687x over previous
import jax
import jax.numpy as jnp
from jax import lax
from jax.experimental import pallas as pl
from jax.experimental.pallas import tpu as pltpu

N_DEV = 4
EPS = 1e-5

_sem_signal = getattr(pltpu, "semaphore_signal", None) or pl.semaphore_signal
_sem_wait = getattr(pltpu, "semaphore_wait", None) or pl.semaphore_wait
_DevId = getattr(pltpu, "DeviceIdType", None) or pl.DeviceIdType

B, H_LOC, W, C = 2, 256, 256, 128
C_OUT = 2 * C
N_GLOBAL = float(N_DEV * H_LOC * W)

HB_STATS = 64
HB_APPLY = 64
NJ_STATS = H_LOC // HB_STATS
NJ_APPLY = H_LOC // HB_APPLY


def _stats_body(x_ref, out_ref, tot_ref, comm_ref, send_sems, recv_sems):
    b = pl.program_id(0)
    j = pl.program_id(1)
    step = b * NJ_STATS + j
    last = B * NJ_STATS - 1

    my = lax.axis_index("i")
    left = lax.rem(my - 1 + N_DEV, N_DEV)
    right = lax.rem(my + 1, N_DEV)

    @pl.when(step == 0)
    def _():
        tot_ref[...] = jnp.zeros_like(tot_ref)
        barrier = pltpu.get_barrier_semaphore()
        for nbr in (left, right):
            _sem_signal(barrier, inc=1, device_id=(nbr,),
                        device_id_type=_DevId.MESH)
        _sem_wait(barrier, 2)

    xb = x_ref[0]
    s = jnp.sum(jnp.sum(xb, axis=0), axis=0)
    ss = jnp.sum(jnp.sum(xb * xb, axis=0), axis=0)

    @pl.when(b == 0)
    def _():
        tot_ref[0, :] += s
        tot_ref[2, :] += ss

    @pl.when(b == 1)
    def _():
        tot_ref[1, :] += s
        tot_ref[3, :] += ss

    @pl.when(step == last)
    def _():
        comm_ref[0] = tot_ref[...]
        rdmas = []
        for d in range(1, N_DEV):
            rdma = pltpu.make_async_remote_copy(
                src_ref=comm_ref.at[0],
                dst_ref=comm_ref.at[d],
                send_sem=send_sems.at[d],
                recv_sem=recv_sems.at[d],
                device_id=(lax.rem(my + d, N_DEV),),
                device_id_type=_DevId.MESH,
            )
            rdma.start()
            rdmas.append(rdma)
        for rdma in rdmas:
            rdma.wait()

        tot = comm_ref[0] + comm_ref[1] + comm_ref[2] + comm_ref[3]
        mean = tot[0:2, :] * (1.0 / N_GLOBAL)
        ex2 = tot[2:4, :] * (1.0 / N_GLOBAL)
        var = ex2 - mean * mean
        rstd = lax.rsqrt(var + EPS)
        out_ref[0:2, :] = mean
        out_ref[2:4, :] = rstd
        out_ref[4:8, :] = jnp.zeros((4, C), jnp.float32)


def _apply_body(x_ref, stats_ref, wp_ref, out_ref):
    b = pl.program_id(0)
    mean = jnp.where(b == 0, stats_ref[0, :], stats_ref[1, :])
    rstd = jnp.where(b == 0, stats_ref[2, :], stats_ref[3, :])

    xb = x_ref[0]
    h = (xb - mean[None, None, :]) * rstd[None, None, :]
    a = h * (0.5 * jnp.tanh(0.5 * h) + 0.5)
    flat = a.reshape(HB_APPLY * W, C).astype(jnp.bfloat16)
    wp = wp_ref[...].astype(jnp.bfloat16)
    out = jnp.dot(flat, wp, preferred_element_type=jnp.float32)
    out_ref[0] = out.reshape(HB_APPLY, W, C_OUT).astype(jnp.bfloat16)


def kernel(x, Wp):
    stats = pl.pallas_call(
        _stats_body,
        grid=(B, NJ_STATS),
        in_specs=[
            pl.BlockSpec((1, HB_STATS, W, C), lambda b, j: (b, j, 0, 0)),
        ],
        out_specs=pl.BlockSpec((8, C), lambda b, j: (0, 0)),
        out_shape=jax.ShapeDtypeStruct((8, C), jnp.float32),
        scratch_shapes=[
            pltpu.VMEM((8, C), jnp.float32),
            pltpu.VMEM((N_DEV, 8, C), jnp.float32),
            pltpu.SemaphoreType.DMA((N_DEV,)),
            pltpu.SemaphoreType.DMA((N_DEV,)),
        ],
        compiler_params=pltpu.CompilerParams(
            dimension_semantics=("arbitrary", "arbitrary"),
            collective_id=0,
        ),
    )(x)

    out = pl.pallas_call(
        _apply_body,
        grid=(B, NJ_APPLY),
        in_specs=[
            pl.BlockSpec((1, HB_APPLY, W, C), lambda b, j: (b, j, 0, 0)),
            pl.BlockSpec((8, C), lambda b, j: (0, 0)),
            pl.BlockSpec((C, C_OUT), lambda b, j: (0, 0)),
        ],
        out_specs=pl.BlockSpec((1, HB_APPLY, W, C_OUT), lambda b, j: (b, j, 0, 0)),
        out_shape=jax.ShapeDtypeStruct((B, H_LOC, W, C_OUT), jnp.bfloat16),
        compiler_params=pltpu.CompilerParams(
            dimension_semantics=("arbitrary", "arbitrary"),
            vmem_limit_bytes=50 * 1024 * 1024,
        ),
    )(x, stats, Wp)
    return out
